# baseline (speedup 1.0000x reference)
import jax
import jax.numpy as jnp
from jax import lax
from jax.experimental import pallas as pl
from jax.experimental.pallas import tpu as pltpu
from jax.experimental.pallas import tpu_sc as plsc

_F = 26
_N = 1000
_R = 16
_B = 16384
_L = 16
_NT = 32
_W = _F * _B // _NT
_GW = _W // _L


def _sc_body(x_hbm, y_hbm, scf_hbm, std_hbm, out_hbm,
             t0_v, t1_v, s0_v, s1_v, x_v, y_v, o_v):
    c = lax.axis_index("c")
    s = lax.axis_index("s")
    tid = s * 2 + c
    start = tid * _W
    f0 = start // _B
    f1 = (start + _W - 1) // _B
    n0 = jnp.minimum(_W, (f0 + 1) * _B - start)

    pltpu.sync_copy(scf_hbm.at[f0], t0_v)
    pltpu.sync_copy(scf_hbm.at[f1], t1_v)
    pltpu.sync_copy(std_hbm.at[f0], s0_v)
    pltpu.sync_copy(std_hbm.at[f1], s1_v)
    pltpu.sync_copy(x_hbm.at[pl.ds(start, _W)], x_v)
    pltpu.sync_copy(y_hbm.at[pl.ds(start, _W)], y_v)

    def _square(i, carry):
        row = t0_v[pl.ds(i * _L, _L)]
        t0_v[pl.ds(i * _L, _L)] = row * row
        row1 = t1_v[pl.ds(i * _L, _L)]
        t1_v[pl.ds(i * _L, _L)] = row1 * row1
        return carry

    lax.fori_loop(0, _N * _R // _L, _square, 0, unroll=4)

    def _make_group(table_v, std_v):
        def _group(g, carry):
            base = g * _L
            xv = x_v[pl.ds(base, _L)]
            yv = y_v[pl.ds(base, _L)]
            acc = [jnp.zeros((_L,), jnp.float32) for _ in range(4)]
            for r in range(_R):
                ax = plsc.load_gather(table_v, [xv + r * _N])
                ay = plsc.load_gather(table_v, [yv + r * _N])
                acc[r % 4] = acc[r % 4] + ax * ay
            sx = plsc.load_gather(std_v, [xv])
            total = (acc[0] + acc[1]) + (acc[2] + acc[3])
            total = jnp.where(xv == yv, total + sx * sx, total)
            o_v[pl.ds(base, _L)] = total
            return carry
        return _group

    g_split = n0 // _L
    lax.fori_loop(0, g_split, _make_group(t0_v, s0_v), 0)
    lax.fori_loop(g_split, _GW, _make_group(t1_v, s1_v), 0)

    pltpu.sync_copy(o_v, out_hbm.at[pl.ds(start, _W)])


@jax.jit
def kernel(x, y, sqrt_covar_factor, std):
    xt = x.astype(jnp.int32).T.reshape(_F * _B)
    yt = y.astype(jnp.int32).T.reshape(_F * _B)
    scf_flat = sqrt_covar_factor.transpose(0, 2, 1).reshape(_F, _R * _N)
    mesh = plsc.VectorSubcoreMesh(core_axis_name="c", subcore_axis_name="s")
    out = pl.kernel(
        _sc_body,
        out_type=jax.ShapeDtypeStruct((_F * _B,), jnp.float32),
        mesh=mesh,
        compiler_params=pltpu.CompilerParams(needs_layout_passes=False),
        scratch_types=[
            pltpu.VMEM((_R * _N,), jnp.float32),
            pltpu.VMEM((_R * _N,), jnp.float32),
            pltpu.VMEM((_N,), jnp.float32),
            pltpu.VMEM((_N,), jnp.float32),
            pltpu.VMEM((_W,), jnp.int32),
            pltpu.VMEM((_W,), jnp.int32),
            pltpu.VMEM((_W,), jnp.float32),
        ],
    )(xt, yt, scf_flat, std)
    return out.reshape(_F, _B).T


# speedup vs baseline: 1.4028x; 1.4028x over previous
import jax
import jax.numpy as jnp
from jax import lax
from jax.experimental import pallas as pl
from jax.experimental.pallas import tpu as pltpu
from jax.experimental.pallas import tpu_sc as plsc

_F = 26
_N = 1000
_R = 16
_B = 16384
_L = 16
_G = _B // _L


def _sc_body(x_hbm, y_hbm, scf_hbm, std_hbm, out_hbm,
             table_v, std_v, x_v, y_v, o_v):
    c = lax.axis_index("c")
    s = lax.axis_index("s")
    f = s * 2 + c

    @pl.when(f < _F)
    def _():
        pltpu.sync_copy(scf_hbm.at[f], table_v)
        pltpu.sync_copy(std_hbm.at[f], std_v)
        pltpu.sync_copy(x_hbm.at[f], x_v)
        pltpu.sync_copy(y_hbm.at[f], y_v)

        @plsc.parallel_loop(0, _N * _R // _L, unroll=4)
        def _square(i):
            row = table_v[pl.ds(i * _L, _L)]
            table_v[pl.ds(i * _L, _L)] = row * row

        @plsc.parallel_loop(0, _G, unroll=2)
        def _group(g):
            base = g * _L
            xv = x_v[pl.ds(base, _L)]
            yv = y_v[pl.ds(base, _L)]
            acc = [jnp.zeros((_L,), jnp.float32) for _ in range(4)]
            for r in range(_R):
                ax = plsc.load_gather(table_v, [xv + r * _N])
                ay = plsc.load_gather(table_v, [yv + r * _N])
                acc[r % 4] = acc[r % 4] + ax * ay
            sx = plsc.load_gather(std_v, [xv])
            total = (acc[0] + acc[1]) + (acc[2] + acc[3])
            total = jnp.where(xv == yv, total + sx * sx, total)
            o_v[pl.ds(base, _L)] = total

        pltpu.sync_copy(o_v, out_hbm.at[f])


@jax.jit
def kernel(x, y, sqrt_covar_factor, std):
    xt = x.astype(jnp.int32).T
    yt = y.astype(jnp.int32).T
    scf_flat = sqrt_covar_factor.transpose(0, 2, 1).reshape(_F, _R * _N)
    mesh = plsc.VectorSubcoreMesh(core_axis_name="c", subcore_axis_name="s")
    out = pl.kernel(
        _sc_body,
        out_type=jax.ShapeDtypeStruct((_F, _B), jnp.float32),
        mesh=mesh,
        compiler_params=pltpu.CompilerParams(needs_layout_passes=False),
        scratch_types=[
            pltpu.VMEM((_R * _N,), jnp.float32),
            pltpu.VMEM((_N,), jnp.float32),
            pltpu.VMEM((_B,), jnp.int32),
            pltpu.VMEM((_B,), jnp.int32),
            pltpu.VMEM((_B,), jnp.float32),
        ],
    )(xt, yt, scf_flat, std)
    return out.T
